# baseline (speedup 1.0000x reference)
import functools

import jax
import jax.numpy as jnp
from jax import lax
from jax.experimental import pallas as pl
from jax.experimental.pallas import tpu as pltpu
from jax.experimental.pallas import tpu_sc as plsc

BATCH = 16384
EMBED_DIM = 32

_info = plsc.get_sparse_core_info()
_NC, _NS = _info.num_cores, _info.num_subcores
_NW = _NC * _NS
_B_PER_W = BATCH // _NW


def _gather_body(users_hbm, items_hbm, user_emb_hbm, item_emb_hbm,
                 urows_hbm, irows_hbm,
                 uidx_v, iidx_v, urows_v, irows_v, usem, isem):
    wid = lax.axis_index("s") * _NC + lax.axis_index("c")
    base = wid * _B_PER_W
    pltpu.sync_copy(users_hbm.at[pl.ds(base, _B_PER_W)], uidx_v)
    pltpu.sync_copy(items_hbm.at[pl.ds(base, _B_PER_W)], iidx_v)
    ucp = pltpu.async_copy(user_emb_hbm.at[uidx_v], urows_v, usem)
    icp = pltpu.async_copy(item_emb_hbm.at[iidx_v], irows_v, isem)
    ucp.wait()
    icp.wait()
    pltpu.sync_copy(urows_v, urows_hbm.at[pl.ds(base, _B_PER_W)])
    pltpu.sync_copy(irows_v, irows_hbm.at[pl.ds(base, _B_PER_W)])


def _sc_gather(users, items, user_emb, item_emb):
    mesh = plsc.VectorSubcoreMesh(core_axis_name="c", subcore_axis_name="s")
    fn = pl.kernel(
        _gather_body,
        mesh=mesh,
        compiler_params=pltpu.CompilerParams(use_tc_tiling_on_sc=False),
        out_type=(
            jax.ShapeDtypeStruct((BATCH, EMBED_DIM), jnp.float32),
            jax.ShapeDtypeStruct((BATCH, EMBED_DIM), jnp.float32),
        ),
        scratch_types=[
            pltpu.VMEM((_B_PER_W,), jnp.int32),
            pltpu.VMEM((_B_PER_W,), jnp.int32),
            pltpu.VMEM((_B_PER_W, EMBED_DIM), jnp.float32),
            pltpu.VMEM((_B_PER_W, EMBED_DIM), jnp.float32),
            pltpu.SemaphoreType.DMA,
            pltpu.SemaphoreType.DMA,
        ],
    )
    return fn(users, items, user_emb, item_emb)


def _dense_body(urows_ref, irows_ref, wu_ref, bu_ref, wi_ref, bi_ref, out_ref):
    uv = jax.lax.dot_general(
        urows_ref[...], wu_ref[...],
        dimension_numbers=(((1,), (1,)), ((), ())),
        preferred_element_type=jnp.float32,
        precision=jax.lax.Precision.HIGHEST,
    ) + bu_ref[...][None, :]
    iv = jax.lax.dot_general(
        irows_ref[...], wi_ref[...],
        dimension_numbers=(((1,), (1,)), ((), ())),
        preferred_element_type=jnp.float32,
        precision=jax.lax.Precision.HIGHEST,
    ) + bi_ref[...][None, :]
    out_ref[...] = jnp.sum(uv * iv, axis=1)


_TC_BLOCK = 2048


def _tc_dense(urows, irows, W_user, b_user, W_item, b_item):
    nblk = BATCH // _TC_BLOCK
    return pl.pallas_call(
        _dense_body,
        grid=(nblk,),
        in_specs=[
            pl.BlockSpec((_TC_BLOCK, EMBED_DIM), lambda i: (i, 0)),
            pl.BlockSpec((_TC_BLOCK, EMBED_DIM), lambda i: (i, 0)),
            pl.BlockSpec((EMBED_DIM, EMBED_DIM), lambda i: (0, 0)),
            pl.BlockSpec((EMBED_DIM,), lambda i: (0,)),
            pl.BlockSpec((EMBED_DIM, EMBED_DIM), lambda i: (0, 0)),
            pl.BlockSpec((EMBED_DIM,), lambda i: (0,)),
        ],
        out_specs=pl.BlockSpec((_TC_BLOCK,), lambda i: (i,)),
        out_shape=jax.ShapeDtypeStruct((BATCH,), jnp.float32),
    )(urows, irows, W_user, b_user, W_item, b_item)


@jax.jit
def kernel(users, items, user_embedding, item_embedding,
           W_user, b_user, W_item, b_item):
    users = users.astype(jnp.int32)
    items = items.astype(jnp.int32)
    urows, irows = _sc_gather(users, items, user_embedding, item_embedding)
    return _tc_dense(urows, irows, W_user, b_user, W_item, b_item)
